# baseline (speedup 1.0000x reference)
import functools

import jax
import jax.numpy as jnp
from jax import lax
from jax.experimental import pallas as pl
from jax.experimental.pallas import tpu as pltpu
from jax.experimental.pallas import tpu_sc as plsc

B, S, D = 16384, 26, 128
K = 5
IDX_LO = 1
NC, NS = 2, 16
NW = NC * NS
RPW = B // NW

_mesh = plsc.VectorSubcoreMesh(core_axis_name="c", subcore_axis_name="s")


@functools.partial(
    pl.kernel,
    mesh=_mesh,
    out_type=jax.ShapeDtypeStruct((B, K, D), jnp.float32),
)
def _gather_copy(x_hbm, out_hbm):
    wid = lax.axis_index("s") * NC + lax.axis_index("c")
    base = wid * RPW
    pltpu.sync_copy(
        x_hbm.at[pl.ds(base, RPW), pl.ds(IDX_LO, K)],
        out_hbm.at[pl.ds(base, RPW)],
    )


def kernel(x):
    return _gather_copy(x)


# speedup vs baseline: 1.0006x; 1.0006x over previous
import functools

import jax
import jax.numpy as jnp
from jax import lax
from jax.experimental import pallas as pl
from jax.experimental.pallas import tpu as pltpu
from jax.experimental.pallas import tpu_sc as plsc

B, S, D = 16384, 26, 128
K = 5
IDX_LO = 1
NC, NS = 2, 16
NW = NC * NS
RPW = B // NW

_mesh = plsc.VectorSubcoreMesh(core_axis_name="c", subcore_axis_name="s")


NSPLIT = 8
CH = RPW // NSPLIT


@functools.partial(
    pl.kernel,
    mesh=_mesh,
    out_type=jax.ShapeDtypeStruct((B, K, D), jnp.float32),
    scratch_types=[pltpu.SemaphoreType.DMA],
)
def _gather_copy(x_hbm, out_hbm, sem):
    wid = lax.axis_index("s") * NC + lax.axis_index("c")
    base = wid * RPW
    copies = [
        pltpu.async_copy(
            x_hbm.at[pl.ds(base + i * CH, CH), pl.ds(IDX_LO, K)],
            out_hbm.at[pl.ds(base + i * CH, CH)],
            sem,
        )
        for i in range(NSPLIT)
    ]
    for c in copies:
        c.wait()


def kernel(x):
    return _gather_copy(x)


# speedup vs baseline: 5.6558x; 5.6526x over previous
import functools

import jax
import jax.numpy as jnp
from jax import lax
from jax.experimental import pallas as pl
from jax.experimental.pallas import tpu as pltpu
from jax.experimental.pallas import tpu_sc as plsc

B, S, D = 16384, 26, 128
K = 5
IDX_LO = 1
NC, NS = 2, 16
NW = NC * NS
RPW = B // NW

_mesh = plsc.VectorSubcoreMesh(core_axis_name="c", subcore_axis_name="s")


NCHUNK = 8
CH = RPW // NCHUNK


@functools.partial(
    pl.kernel,
    mesh=_mesh,
    out_type=jax.ShapeDtypeStruct((B, K, D), jnp.float32),
    scratch_types=[
        pltpu.VMEM((CH, K, D), jnp.float32),
        pltpu.VMEM((CH, K, D), jnp.float32),
        pltpu.SemaphoreType.DMA,
        pltpu.SemaphoreType.DMA,
        pltpu.SemaphoreType.DMA,
        pltpu.SemaphoreType.DMA,
    ],
)
def _gather_copy(x_hbm, out_hbm, buf0, buf1, ls0, ls1, ss0, ss1):
    wid = lax.axis_index("s") * NC + lax.axis_index("c")
    base = wid * RPW
    bufs, lsems, ssems = (buf0, buf1), (ls0, ls1), (ss0, ss1)

    def start_load(i):
        return pltpu.async_copy(
            x_hbm.at[pl.ds(base + i * CH, CH), pl.ds(IDX_LO, K)],
            bufs[i % 2],
            lsems[i % 2],
        )

    loads = [None] * NCHUNK
    stores = [None] * NCHUNK
    loads[0] = start_load(0)
    for i in range(NCHUNK):
        if i + 1 < NCHUNK:
            if i >= 1:
                stores[i - 1].wait()
            loads[i + 1] = start_load(i + 1)
        loads[i].wait()
        stores[i] = pltpu.async_copy(
            bufs[i % 2],
            out_hbm.at[pl.ds(base + i * CH, CH)],
            ssems[i % 2],
        )
    stores[NCHUNK - 2].wait()
    stores[NCHUNK - 1].wait()


def kernel(x):
    return _gather_copy(x)
